# baseline (speedup 1.0000x reference)
import math

import numpy as np
import jax
import jax.numpy as jnp
from jax.experimental import pallas as pl
from jax.experimental.pallas import tpu as pltpu
from jax.experimental.pallas import tpu_sc as plsc

N = 100000
B = 8192
NBR = 64
IN = 128
EF = 16
TF = 32
H = 128
OUT = 128
HEADS = 2
DS = H + EF + TF
KP = NBR * EF

BS = 256
GW = 256

_PERM = np.concatenate([
    np.arange(H),
    H + 2 * np.arange(TF // 2),
    H + 1 + 2 * np.arange(TF // 2),
    np.arange(H + TF, DS),
])

_J = np.arange(KP)
_R_NP = (_J[None, :] // EF == np.arange(NBR)[:, None]).astype(np.float32)
_T_NP = (_J[None, :] % EF == np.arange(EF)[:, None]).astype(np.float32)


def _sc_gather(x, idx):
    num = idx.shape[0]
    idx2 = idx.reshape(1, num)
    mesh = plsc.VectorSubcoreMesh(core_axis_name="core", subcore_axis_name="subcore")

    @pl.kernel(out_type=jax.ShapeDtypeStruct((num, x.shape[1]), x.dtype),
               mesh=mesh)
    def gather_kernel(x_hbm, i_hbm, o_hbm):
        def body(i_vmem, o_vmem):
            pltpu.sync_copy(x_hbm.at[i_vmem.at[0]], o_vmem)

        pltpu.emit_pipeline(
            body,
            grid=(num // GW,),
            in_specs=[pl.BlockSpec((1, GW), lambda i: (0, i))],
            out_specs=[pl.BlockSpec((GW, x.shape[1]), lambda i: (i, 0))],
            core_axis_name=("core", "subcore"),
            dimension_semantics=(pltpu.PARALLEL,),
        )(i_hbm, o_hbm)

    return gather_kernel(x, idx2)


def _fused_body(src_ref, nf_ref, ts_ref, relsp_ref, suffix_ref, wtt_ref,
                win_ref, bin_ref, wqk0_ref, wqk1_ref, wvfcf0_ref, wvfct0_ref,
                wvfcf1_ref, wvfct1_ref, bfc_ref, lng_ref, lnb_ref,
                wm1a_ref, wm1b_ref, bm1_ref, wm2_ref, bm2_ref, wout_ref,
                bout_ref, r_ref, rt_ref, t_ref, tt_ref, out_ref):
    f32 = jnp.float32
    win = win_ref[...]
    bin_ = bin_ref[...]

    src_x = jnp.maximum(
        jnp.dot(src_ref[...], win, preferred_element_type=f32) + bin_, 0.0)
    q_in = jnp.concatenate(
        [src_x, jnp.broadcast_to(suffix_ref[...], (BS, DS - H))], axis=1)

    nf2 = nf_ref[...].reshape(BS * NBR, IN)
    nfy = jnp.maximum(jnp.dot(nf2, win, preferred_element_type=f32) + bin_, 0.0)
    nfy3 = nfy.reshape(BS, NBR, H)

    ts = ts_ref[...]
    tsr = jnp.max(ts, axis=1, keepdims=True) - ts
    tsr_rep = jnp.dot(tsr, r_ref[...], preferred_element_type=f32)
    h_p = tsr_rep * wtt_ref[...]
    sin_p = jnp.sin(h_p) * 0.25
    cos_p = jnp.cos(h_p) * 0.25
    rels_p = relsp_ref[...]

    o = jnp.broadcast_to(bfc_ref[...], (BS, DS))
    head_refs = ((wqk0_ref, wvfcf0_ref, wvfct0_ref),
                 (wqk1_ref, wvfcf1_ref, wvfct1_ref))
    for wqk_ref, wvfcf_ref, wvfct_ref in head_refs:
        u = jnp.dot(q_in, wqk_ref[...], preferred_element_type=f32)
        u_f = u[:, :H]
        t_mat = t_ref[...]
        us = jnp.dot(u[:, H:H + EF], t_mat, preferred_element_type=f32)
        uc = jnp.dot(u[:, H + EF:H + TF], t_mat, preferred_element_type=f32)
        ur = jnp.dot(u[:, H + TF:], t_mat, preferred_element_type=f32)
        prod_t = sin_p * us + cos_p * uc + rels_p * ur
        logits = (jnp.sum(nfy3 * u_f[:, None, :], axis=-1)
                  + jnp.dot(prod_t, rt_ref[...], preferred_element_type=f32))
        e = jnp.exp(logits)
        e_rep = jnp.dot(e, r_ref[...], preferred_element_type=f32)
        rcp = float(EF) / jnp.sum(e_rep, axis=-1, keepdims=True)
        agg_f = jnp.sum(nfy3 * e[:, :, None], axis=1)
        tt_mat = tt_ref[...]
        agg_s = jnp.dot(e_rep * sin_p, tt_mat, preferred_element_type=f32)
        agg_c = jnp.dot(e_rep * cos_p, tt_mat, preferred_element_type=f32)
        agg_r = jnp.dot(e_rep * rels_p, tt_mat, preferred_element_type=f32)
        agg_t = jnp.concatenate([agg_s, agg_c, agg_r], axis=1)
        o = (o + (jnp.dot(agg_f, wvfcf_ref[...], preferred_element_type=f32)
                  + jnp.dot(agg_t, wvfct_ref[...], preferred_element_type=f32))
             * rcp)

    val = o + q_in
    m = jnp.mean(val, axis=-1, keepdims=True)
    v = jnp.mean((val - m) * (val - m), axis=-1, keepdims=True)
    val = (val - m) * jax.lax.rsqrt(v + 1e-5) * lng_ref[...] + lnb_ref[...]

    h1 = jnp.maximum(
        jnp.dot(src_x, wm1a_ref[...], preferred_element_type=f32)
        + jnp.dot(val, wm1b_ref[...], preferred_element_type=f32)
        + bm1_ref[...], 0.0)
    h2 = jnp.dot(h1, wm2_ref[...], preferred_element_type=f32) + bm2_ref[...]
    out_ref[...] = (jnp.dot(h2, wout_ref[...], preferred_element_type=f32)
                    + bout_ref[...])


def _full(shape):
    return pl.BlockSpec(shape, lambda i: (0,) * len(shape))


CH = 4
BC = B // CH


def kernel(x, batch, neighbors, ts, rels, W_in, b_in, w_t, b_t, W_q, W_k, W_v,
           W_fc, b_fc, ln_g, ln_b, W_m1, b_m1, W_m2, b_m2, W_out, b_out,
           src_param):
    perm = _PERM
    relsp = rels.reshape(B, KP)

    scale = 1.0 / math.sqrt(DS)
    Wq_p, Wk_p, Wv_p = W_q[perm], W_k[perm], W_v[perm]
    Wfc_p = W_fc[:, perm]
    wqk = [Wq_p[:, h * DS:(h + 1) * DS] @ Wk_p[:, h * DS:(h + 1) * DS].T * scale
           for h in range(HEADS)]
    wvfc = [Wv_p[:, h * DS:(h + 1) * DS] @ Wfc_p[h * DS:(h + 1) * DS]
            for h in range(HEADS)]
    suffix = jnp.concatenate(
        [jnp.sin(b_t) * 0.25, jnp.cos(b_t) * 0.25, src_param[0]]).reshape(1, DS - H)
    wtt = jnp.tile(w_t.reshape(1, EF), (1, NBR))

    weight_args = (
        W_in, b_in.reshape(1, H), wqk[0], wqk[1],
        wvfc[0][:H], wvfc[0][H:], wvfc[1][:H], wvfc[1][H:],
        b_fc[perm].reshape(1, DS), ln_g[perm].reshape(1, DS),
        ln_b[perm].reshape(1, DS), W_m1[:H], W_m1[H:][perm],
        b_m1.reshape(1, H), W_m2, b_m2.reshape(1, H), W_out,
        b_out.reshape(1, OUT),
        jnp.asarray(_R_NP), jnp.asarray(_R_NP.T),
        jnp.asarray(_T_NP), jnp.asarray(_T_NP.T),
    )
    weight_specs = [
        _full((IN, H)),
        _full((1, H)),
        _full((DS, DS)), _full((DS, DS)),
        _full((H, DS)), _full((DS - H, DS)),
        _full((H, DS)), _full((DS - H, DS)),
        _full((1, DS)),
        _full((1, DS)), _full((1, DS)),
        _full((H, H)),
        _full((DS, H)),
        _full((1, H)),
        _full((H, H)), _full((1, H)),
        _full((H, OUT)), _full((1, OUT)),
        _full((NBR, KP)),
        _full((KP, NBR)),
        _full((EF, KP)),
        _full((KP, EF)),
    ]

    gathered = []
    for c in range(CH):
        lo = c * BC
        src_c = _sc_gather(x, jax.lax.dynamic_slice_in_dim(batch, lo, BC)
                           .astype(jnp.int32))
        nf_c = _sc_gather(
            x, jax.lax.dynamic_slice_in_dim(neighbors, lo, BC)
            .reshape(-1).astype(jnp.int32))
        gathered.append((src_c, nf_c))

    outs = []
    for c in range(CH):
        src_c, nf_c = gathered[c]
        nf3_c = nf_c.reshape(BC, NBR, IN)
        off = (c * BC) // BS
        out_c = pl.pallas_call(
            _fused_body,
            grid=(BC // BS,),
            in_specs=[
                pl.BlockSpec((BS, IN), lambda i: (i, 0)),
                pl.BlockSpec((BS, NBR, IN), lambda i: (i, 0, 0)),
                pl.BlockSpec((BS, NBR), lambda i, o=off: (o + i, 0)),
                pl.BlockSpec((BS, KP), lambda i, o=off: (o + i, 0)),
                _full((1, DS - H)),
                _full((1, KP)),
            ] + weight_specs,
            out_specs=pl.BlockSpec((BS, OUT), lambda i: (i, 0)),
            out_shape=jax.ShapeDtypeStruct((BC, OUT), jnp.float32),
            compiler_params=pltpu.CompilerParams(
                dimension_semantics=("parallel",)),
        )(src_c, nf3_c, ts, relsp, suffix, wtt, *weight_args)
        outs.append(out_c)
    return jnp.concatenate(outs, axis=0)


# speedup vs baseline: 1.1436x; 1.1436x over previous
import math

import numpy as np
import jax
import jax.numpy as jnp
from jax.experimental import pallas as pl
from jax.experimental.pallas import tpu as pltpu
from jax.experimental.pallas import tpu_sc as plsc

N = 100000
B = 8192
NBR = 64
IN = 128
EF = 16
TF = 32
H = 128
OUT = 128
HEADS = 2
DS = H + EF + TF
KP = NBR * EF

BS = 256
GW = 128

_PERM = np.concatenate([
    np.arange(H),
    H + 2 * np.arange(TF // 2),
    H + 1 + 2 * np.arange(TF // 2),
    np.arange(H + TF, DS),
])

_J = np.arange(KP)
_R_NP = (_J[None, :] // EF == np.arange(NBR)[:, None]).astype(np.float32)
_T_NP = (_J[None, :] % EF == np.arange(EF)[:, None]).astype(np.float32)


def _sc_gather(x, idx):
    num = idx.shape[0]
    idx2 = idx.reshape(1, num)
    mesh = plsc.VectorSubcoreMesh(core_axis_name="core", subcore_axis_name="subcore")

    @pl.kernel(out_type=jax.ShapeDtypeStruct((num, x.shape[1]), x.dtype),
               mesh=mesh)
    def gather_kernel(x_hbm, i_hbm, o_hbm):
        def body(i_vmem, o_vmem):
            pltpu.sync_copy(x_hbm.at[i_vmem.at[0]], o_vmem)

        pltpu.emit_pipeline(
            body,
            grid=(num // GW,),
            in_specs=[pl.BlockSpec((1, GW), lambda i: (0, i))],
            out_specs=[pl.BlockSpec((GW, x.shape[1]), lambda i: (i, 0))],
            core_axis_name=("core", "subcore"),
            dimension_semantics=(pltpu.PARALLEL,),
        )(i_hbm, o_hbm)

    return gather_kernel(x, idx2)


def _fused_body(src_ref, nf_ref, ts_ref, relsp_ref, suffix_ref, wtt_ref,
                win_ref, bin_ref, wqk0_ref, wqk1_ref, wvfcf0_ref, wvfct0_ref,
                wvfcf1_ref, wvfct1_ref, bfc_ref, lng_ref, lnb_ref,
                wm1a_ref, wm1b_ref, bm1_ref, wm2_ref, bm2_ref, wout_ref,
                bout_ref, r_ref, rt_ref, t_ref, tt_ref, out_ref):
    f32 = jnp.float32
    win = win_ref[...]
    bin_ = bin_ref[...]

    src_x = jnp.maximum(
        jnp.dot(src_ref[...], win, preferred_element_type=f32) + bin_, 0.0)
    q_in = jnp.concatenate(
        [src_x, jnp.broadcast_to(suffix_ref[...], (BS, DS - H))], axis=1)

    nf2 = nf_ref[...].reshape(BS * NBR, IN)
    nfy = jnp.maximum(jnp.dot(nf2, win, preferred_element_type=f32) + bin_, 0.0)
    nfy3 = nfy.reshape(BS, NBR, H)

    ts = ts_ref[...]
    tsr = jnp.max(ts, axis=1, keepdims=True) - ts
    tsr_rep = jnp.dot(tsr, r_ref[...], preferred_element_type=f32)
    h_p = tsr_rep * wtt_ref[...]
    k = jnp.round(h_p * 0.15915494309189535)
    r = (h_p - k * 6.28125) - k * 1.9353071795864769e-3
    r2 = r * r
    sin_p = r * (0.25 + r2 * (-4.1666668e-2 + r2 * (2.0833334e-3 + r2 * (
        -4.9603175e-5 + r2 * (6.8893005e-7 + r2 * (-6.2630286e-9
                                                   + r2 * 4.0147856e-11))))))
    cos_p = 0.25 + r2 * (-0.125 + r2 * (1.0416667e-2 + r2 * (
        -3.4722222e-4 + r2 * (6.2003968e-6 + r2 * (-6.8893005e-8
                                                   + r2 * (5.2192065e-10
                                                           + r2 * -2.8677206e-12))))))
    rels_p = relsp_ref[...]

    o = jnp.broadcast_to(bfc_ref[...], (BS, DS))
    head_refs = ((wqk0_ref, wvfcf0_ref, wvfct0_ref),
                 (wqk1_ref, wvfcf1_ref, wvfct1_ref))
    for wqk_ref, wvfcf_ref, wvfct_ref in head_refs:
        u = jnp.dot(q_in, wqk_ref[...], preferred_element_type=f32)
        u_f = u[:, :H]
        t_mat = t_ref[...]
        us = jnp.dot(u[:, H:H + EF], t_mat, preferred_element_type=f32)
        uc = jnp.dot(u[:, H + EF:H + TF], t_mat, preferred_element_type=f32)
        ur = jnp.dot(u[:, H + TF:], t_mat, preferred_element_type=f32)
        prod_t = sin_p * us + cos_p * uc + rels_p * ur
        logits = (jnp.sum(nfy3 * u_f[:, None, :], axis=-1)
                  + jnp.dot(prod_t, rt_ref[...], preferred_element_type=f32))
        e = jnp.exp(logits)
        e_rep = jnp.dot(e, r_ref[...], preferred_element_type=f32)
        rcp = float(EF) / jnp.sum(e_rep, axis=-1, keepdims=True)
        agg_f = jnp.sum(nfy3 * e[:, :, None], axis=1)
        tt_mat = tt_ref[...]
        agg_s = jnp.dot(e_rep * sin_p, tt_mat, preferred_element_type=f32)
        agg_c = jnp.dot(e_rep * cos_p, tt_mat, preferred_element_type=f32)
        agg_r = jnp.dot(e_rep * rels_p, tt_mat, preferred_element_type=f32)
        agg_t = jnp.concatenate([agg_s, agg_c, agg_r], axis=1)
        o = (o + (jnp.dot(agg_f, wvfcf_ref[...], preferred_element_type=f32)
                  + jnp.dot(agg_t, wvfct_ref[...], preferred_element_type=f32))
             * rcp)

    val = o + q_in
    m = jnp.mean(val, axis=-1, keepdims=True)
    v = jnp.mean((val - m) * (val - m), axis=-1, keepdims=True)
    val = (val - m) * jax.lax.rsqrt(v + 1e-5) * lng_ref[...] + lnb_ref[...]

    h1 = jnp.maximum(
        jnp.dot(src_x, wm1a_ref[...], preferred_element_type=f32)
        + jnp.dot(val, wm1b_ref[...], preferred_element_type=f32)
        + bm1_ref[...], 0.0)
    h2 = jnp.dot(h1, wm2_ref[...], preferred_element_type=f32) + bm2_ref[...]
    out_ref[...] = (jnp.dot(h2, wout_ref[...], preferred_element_type=f32)
                    + bout_ref[...])


def _full(shape):
    return pl.BlockSpec(shape, lambda i: (0,) * len(shape))


CH = 4
BC = B // CH


def kernel(x, batch, neighbors, ts, rels, W_in, b_in, w_t, b_t, W_q, W_k, W_v,
           W_fc, b_fc, ln_g, ln_b, W_m1, b_m1, W_m2, b_m2, W_out, b_out,
           src_param):
    perm = _PERM
    relsp = rels.reshape(B, KP)

    scale = 1.0 / math.sqrt(DS)
    Wq_p, Wk_p, Wv_p = W_q[perm], W_k[perm], W_v[perm]
    Wfc_p = W_fc[:, perm]
    wqk = [Wq_p[:, h * DS:(h + 1) * DS] @ Wk_p[:, h * DS:(h + 1) * DS].T * scale
           for h in range(HEADS)]
    wvfc = [Wv_p[:, h * DS:(h + 1) * DS] @ Wfc_p[h * DS:(h + 1) * DS]
            for h in range(HEADS)]
    suffix = jnp.concatenate(
        [jnp.sin(b_t) * 0.25, jnp.cos(b_t) * 0.25, src_param[0]]).reshape(1, DS - H)
    wtt = jnp.tile(w_t.reshape(1, EF), (1, NBR))

    weight_args = (
        W_in, b_in.reshape(1, H), wqk[0], wqk[1],
        wvfc[0][:H], wvfc[0][H:], wvfc[1][:H], wvfc[1][H:],
        b_fc[perm].reshape(1, DS), ln_g[perm].reshape(1, DS),
        ln_b[perm].reshape(1, DS), W_m1[:H], W_m1[H:][perm],
        b_m1.reshape(1, H), W_m2, b_m2.reshape(1, H), W_out,
        b_out.reshape(1, OUT),
        jnp.asarray(_R_NP), jnp.asarray(_R_NP.T),
        jnp.asarray(_T_NP), jnp.asarray(_T_NP.T),
    )
    weight_specs = [
        _full((IN, H)),
        _full((1, H)),
        _full((DS, DS)), _full((DS, DS)),
        _full((H, DS)), _full((DS - H, DS)),
        _full((H, DS)), _full((DS - H, DS)),
        _full((1, DS)),
        _full((1, DS)), _full((1, DS)),
        _full((H, H)),
        _full((DS, H)),
        _full((1, H)),
        _full((H, H)), _full((1, H)),
        _full((H, OUT)), _full((1, OUT)),
        _full((NBR, KP)),
        _full((KP, NBR)),
        _full((EF, KP)),
        _full((KP, EF)),
    ]

    gathered = []
    for c in range(CH):
        lo = c * BC
        src_c = _sc_gather(x, jax.lax.dynamic_slice_in_dim(batch, lo, BC)
                           .astype(jnp.int32))
        nf_c = _sc_gather(
            x, jax.lax.dynamic_slice_in_dim(neighbors, lo, BC)
            .reshape(-1).astype(jnp.int32))
        gathered.append((src_c, nf_c))

    outs = []
    for c in range(CH):
        src_c, nf_c = gathered[c]
        nf3_c = nf_c.reshape(BC, NBR, IN)
        off = (c * BC) // BS
        out_c = pl.pallas_call(
            _fused_body,
            grid=(BC // BS,),
            in_specs=[
                pl.BlockSpec((BS, IN), lambda i: (i, 0)),
                pl.BlockSpec((BS, NBR, IN), lambda i: (i, 0, 0)),
                pl.BlockSpec((BS, NBR), lambda i, o=off: (o + i, 0)),
                pl.BlockSpec((BS, KP), lambda i, o=off: (o + i, 0)),
                _full((1, DS - H)),
                _full((1, KP)),
            ] + weight_specs,
            out_specs=pl.BlockSpec((BS, OUT), lambda i: (i, 0)),
            out_shape=jax.ShapeDtypeStruct((BC, OUT), jnp.float32),
            compiler_params=pltpu.CompilerParams(
                dimension_semantics=("parallel",)),
        )(src_c, nf3_c, ts, relsp, suffix, wtt, *weight_args)
        outs.append(out_c)
    return jnp.concatenate(outs, axis=0)
